# baseline (speedup 1.0000x reference)
import jax
import jax.numpy as jnp
from jax.experimental import pallas as pl
from jax.experimental.pallas import tpu as pltpu

DIM = 128
N_CHUNKS = 16


def _copy_body(x_hbm, o_hbm, sems):
    rows = x_hbm.shape[0]
    chunk = rows // N_CHUNKS
    for i in range(N_CHUNKS):
        pltpu.make_async_copy(
            x_hbm.at[pl.ds(i * chunk, chunk)],
            o_hbm.at[pl.ds(i * chunk, chunk)],
            sems.at[i],
        ).start()
    for i in range(N_CHUNKS):
        pltpu.make_async_copy(
            x_hbm.at[pl.ds(i * chunk, chunk)],
            o_hbm.at[pl.ds(i * chunk, chunk)],
            sems.at[i],
        ).wait()


def kernel(x):
    x = x.astype(jnp.float32)
    n = x.size // DIM
    x = x.reshape(n, DIM)
    return pl.pallas_call(
        _copy_body,
        out_shape=jax.ShapeDtypeStruct((n, DIM), jnp.float32),
        in_specs=[pl.BlockSpec(memory_space=pltpu.MemorySpace.HBM)],
        out_specs=pl.BlockSpec(memory_space=pltpu.MemorySpace.HBM),
        scratch_shapes=[
            pltpu.SemaphoreType.DMA((N_CHUNKS,)),
        ],
    )(x)


# speedup vs baseline: 15.9160x; 15.9160x over previous
import functools

import jax
import jax.numpy as jnp
from jax import lax
from jax.experimental import pallas as pl
from jax.experimental.pallas import tpu as pltpu
from jax.experimental.pallas import tpu_sc as plsc

DIM = 128
ROWS = 32768
NC = 2
NS = 16
NW = NC * NS
W_ROWS = ROWS // NW
CHUNKS = 4
C_ROWS = W_ROWS // CHUNKS
NBUF = 2

_mesh = plsc.VectorSubcoreMesh(
    core_axis_name="c", subcore_axis_name="s", num_cores=NC, num_subcores=NS
)


@functools.partial(
    pl.kernel,
    out_type=jax.ShapeDtypeStruct((ROWS, DIM), jnp.float32),
    mesh=_mesh,
    scratch_types=[
        pltpu.VMEM((NBUF, C_ROWS, DIM), jnp.float32),
        pltpu.SemaphoreType.DMA((NBUF,)),
        pltpu.SemaphoreType.DMA((NBUF,)),
    ],
)
def _sc_copy(x_hbm, o_hbm, buf, in_sems, out_sems):
    wid = lax.axis_index("s") * NC + lax.axis_index("c")
    base = wid * W_ROWS

    def read(j):
        return pltpu.make_async_copy(
            x_hbm.at[pl.ds(base + j * C_ROWS, C_ROWS)], buf.at[j % NBUF],
            in_sems.at[j % NBUF],
        )

    def write(j):
        return pltpu.make_async_copy(
            buf.at[j % NBUF], o_hbm.at[pl.ds(base + j * C_ROWS, C_ROWS)],
            out_sems.at[j % NBUF],
        )

    for j in range(NBUF):
        read(j).start()
    for j in range(CHUNKS):
        read(j).wait()
        write(j).start()
        nxt = j + NBUF
        if nxt < CHUNKS:
            write(nxt - NBUF).wait()
            read(nxt).start()
    for j in range(CHUNKS - NBUF, CHUNKS):
        write(j).wait()


def kernel(x):
    x = x.astype(jnp.float32).reshape(ROWS, DIM)
    return _sc_copy(x)


# speedup vs baseline: 42.6295x; 2.6784x over previous
import jax
import jax.numpy as jnp
from jax.experimental import pallas as pl
from jax.experimental.pallas import tpu as pltpu

DIM = 128
CHUNK_ROWS = (2048, 2048, 4096, 8192, 16384)
N_CHUNKS = len(CHUNK_ROWS)
CHUNK_OFFS = tuple(sum(CHUNK_ROWS[:i]) for i in range(N_CHUNKS))


def _copy_body(x_hbm, o_hbm, *bufs_and_sems):
    bufs = bufs_and_sems[:N_CHUNKS]
    in_sems, out_sems = bufs_and_sems[N_CHUNKS], bufs_and_sems[N_CHUNKS + 1]

    def read(i):
        return pltpu.make_async_copy(
            x_hbm.at[pl.ds(CHUNK_OFFS[i], CHUNK_ROWS[i])], bufs[i], in_sems.at[i]
        )

    def write(i):
        return pltpu.make_async_copy(
            bufs[i], o_hbm.at[pl.ds(CHUNK_OFFS[i], CHUNK_ROWS[i])], out_sems.at[i]
        )

    for i in range(N_CHUNKS):
        read(i).start()
    for i in range(N_CHUNKS):
        read(i).wait()
        write(i).start()
    for i in range(N_CHUNKS):
        write(i).wait()


def kernel(x):
    x = x.astype(jnp.float32)
    n = x.size // DIM
    x = x.reshape(n, DIM)
    return pl.pallas_call(
        _copy_body,
        out_shape=jax.ShapeDtypeStruct((n, DIM), jnp.float32),
        in_specs=[pl.BlockSpec(memory_space=pltpu.MemorySpace.HBM)],
        out_specs=pl.BlockSpec(memory_space=pltpu.MemorySpace.HBM),
        scratch_shapes=[
            *[pltpu.VMEM((r, DIM), jnp.float32) for r in CHUNK_ROWS],
            pltpu.SemaphoreType.DMA((N_CHUNKS,)),
            pltpu.SemaphoreType.DMA((N_CHUNKS,)),
        ],
    )(x)


# speedup vs baseline: 47.7430x; 1.1200x over previous
import jax
import jax.numpy as jnp
from jax.experimental import pallas as pl
from jax.experimental.pallas import tpu as pltpu

DIM = 128
CHUNK_ROWS = (16384, 16384)
N_CHUNKS = len(CHUNK_ROWS)
CHUNK_OFFS = tuple(sum(CHUNK_ROWS[:i]) for i in range(N_CHUNKS))


def _copy_body(x_hbm, o_hbm, *bufs_and_sems):
    bufs = bufs_and_sems[:N_CHUNKS]
    in_sems, out_sems = bufs_and_sems[N_CHUNKS], bufs_and_sems[N_CHUNKS + 1]

    def read(i):
        return pltpu.make_async_copy(
            x_hbm.at[pl.ds(CHUNK_OFFS[i], CHUNK_ROWS[i])], bufs[i], in_sems.at[i]
        )

    def write(i):
        return pltpu.make_async_copy(
            bufs[i], o_hbm.at[pl.ds(CHUNK_OFFS[i], CHUNK_ROWS[i])], out_sems.at[i]
        )

    for i in range(N_CHUNKS):
        read(i).start()
    for i in range(N_CHUNKS):
        read(i).wait()
        write(i).start()
    for i in range(N_CHUNKS):
        write(i).wait()


def kernel(x):
    x = x.astype(jnp.float32)
    n = x.size // DIM
    x = x.reshape(n, DIM)
    return pl.pallas_call(
        _copy_body,
        out_shape=jax.ShapeDtypeStruct((n, DIM), jnp.float32),
        in_specs=[pl.BlockSpec(memory_space=pltpu.MemorySpace.HBM)],
        out_specs=pl.BlockSpec(memory_space=pltpu.MemorySpace.HBM),
        scratch_shapes=[
            *[pltpu.VMEM((r, DIM), jnp.float32) for r in CHUNK_ROWS],
            pltpu.SemaphoreType.DMA((N_CHUNKS,)),
            pltpu.SemaphoreType.DMA((N_CHUNKS,)),
        ],
    )(x)
